# baseline (speedup 1.0000x reference)
import functools

import jax
import jax.numpy as jnp
from jax import lax
from jax.experimental import pallas as pl
from jax.experimental.pallas import tpu as pltpu
from jax.experimental.pallas import tpu_sc as plsc

_D = 128
_N = 10000
_L = 16
_NW = 32
_NPAD = 10240
_TN = _NPAD // _NW
_EB = 4000
_NEG = -3.4028235e38

_GDN = lax.GatherDimensionNumbers(
    offset_dims=(), collapsed_slice_dims=(0,), start_index_map=(0,))


def _take(v, j):
    return lax.gather(v, j[:, None], _GDN, (1,),
                      mode=lax.GatherScatterMode.PROMISE_IN_BOUNDS)


def _mv_body(in_ref, w_ref, b_ref, o_ref):
    blk = in_ref[0]
    y = lax.dot_general(w_ref[...], blk, (((1,), (1,)), ((), ())),
                        preferred_element_type=jnp.float32)
    y = y + b_ref[0, 0]
    o_ref[0] = jnp.where(y >= 0, y, 0.2 * y)


def _matvec(x3, W, b2):
    nblk = x3.shape[0]
    return pl.pallas_call(
        _mv_body,
        grid=(nblk,),
        in_specs=[
            pl.BlockSpec((1, _EB, _D), lambda i: (i, 0, 0)),
            pl.BlockSpec((1, _D), lambda i: (0, 0)),
            pl.BlockSpec((1, 1), lambda i: (0, 0)),
        ],
        out_specs=pl.BlockSpec((1, 1, _EB), lambda i: (i, 0, 0)),
        out_shape=jax.ShapeDtypeStruct((nblk, 1, _EB), jnp.float32),
    )(x3, W, b2)


def _wid():
    return lax.axis_index("s") * 2 + lax.axis_index("c")


def _mesh():
    return plsc.VectorSubcoreMesh(core_axis_name="c", subcore_axis_name="s")


_SC_PARAMS = pltpu.CompilerParams(needs_layout_passes=False)


def _make_seg_stats(E):
    ch = E // _NW
    nv = ch // _L

    @functools.partial(
        pl.kernel,
        mesh=_mesh(),
        compiler_params=_SC_PARAMS,
        out_type=(
            jax.ShapeDtypeStruct((_NW * _NPAD,), jnp.float32),
            jax.ShapeDtypeStruct((_NW * _NPAD,), jnp.float32),
        ),
        scratch_types=[
            pltpu.VMEM((ch,), jnp.float32),
            pltpu.VMEM((ch,), jnp.int32),
            pltpu.VMEM((_NPAD,), jnp.float32),
            pltpu.VMEM((_NPAD,), jnp.float32),
        ],
    )
    def k(x_hbm, idx_hbm, pmax_hbm, psum_hbm, xv, iv, lm, ls):
        wid = _wid()
        base = wid * ch
        pltpu.sync_copy(x_hbm.at[pl.ds(base, ch)], xv)
        pltpu.sync_copy(idx_hbm.at[pl.ds(base, ch)], iv)
        neg = jnp.full((_L,), _NEG, jnp.float32)
        zero = jnp.zeros((_L,), jnp.float32)

        @plsc.parallel_loop(0, _NPAD // _L, 1, unroll=4)
        def init(i):
            lm[pl.ds(i * _L, _L)] = neg
            ls[pl.ds(i * _L, _L)] = zero

        iota = lax.iota(jnp.int32, _L)

        def body(i, c):
            s = i * _L
            xb = xv[pl.ds(s, _L)]
            ib = iv[pl.ds(s, _L)]
            v = xb
            for k_ in (1, 2, 4, 8):
                j = jnp.maximum(iota - k_, 0)
                sh_i = _take(ib, j)
                sh_v = _take(v, j)
                v = jnp.where(sh_i == ib, jnp.maximum(v, sh_v), v)
            nxt = _take(ib, jnp.minimum(iota + 1, _L - 1))
            is_last = (iota == _L - 1) | (ib != nxt)
            z = jnp.where(is_last, (_L - 1) - iota, 0)
            rl = (_L - 1) - lax.rev(plsc.cummax(lax.rev(z, (0,))), (0,))
            m_run = _take(v, rl)
            e = jnp.exp(xb - m_run)
            cs = plsc.cumsum(e)
            cx = cs - e
            prv = _take(ib, jnp.maximum(iota - 1, 0))
            is_start = (iota == 0) | (ib != prv)
            rs = plsc.cummax(jnp.where(is_start, iota, 0))
            run = cs - _take(cx, rs)
            msk = rl == iota
            cur_m = plsc.load_gather(lm, [ib])
            cur_s = plsc.load_gather(ls, [ib])
            nm = jnp.maximum(cur_m, m_run)
            ns = (cur_s * jnp.exp(jnp.maximum(cur_m - nm, -100.0))
                  + run * jnp.exp(m_run - nm))
            plsc.store_scatter(lm, [ib], nm, mask=msk)
            plsc.store_scatter(ls, [ib], ns, mask=msk)
            return c
        lax.fori_loop(0, nv, body, 0, unroll=2)
        pltpu.sync_copy(lm, pmax_hbm.at[pl.ds(wid * _NPAD, _NPAD)])
        pltpu.sync_copy(ls, psum_hbm.at[pl.ds(wid * _NPAD, _NPAD)])

    return k


def _make_combine(nparts):
    R = nparts * _NW

    @functools.partial(
        pl.kernel,
        mesh=_mesh(),
        compiler_params=_SC_PARAMS,
        out_type=(
            jax.ShapeDtypeStruct((_NPAD,), jnp.float32),
            jax.ShapeDtypeStruct((_NPAD,), jnp.float32),
        ),
        scratch_types=[
            pltpu.VMEM((R * _TN,), jnp.float32),
            pltpu.VMEM((R * _TN,), jnp.float32),
            pltpu.VMEM((_TN,), jnp.float32),
            pltpu.VMEM((_TN,), jnp.float32),
            pltpu.SemaphoreType.DMA,
        ],
    )
    def k(*args):
        (pm_hbms, ps_hbms) = (args[0:2 * nparts:2], args[1:2 * nparts:2])
        gm_hbm, gi_hbm, bm, bs, om, og, sem = args[2 * nparts:]
        wid = _wid()
        off = wid * _TN
        copies = []
        for p in range(nparts):
            for t in range(_NW):
                r = p * _NW + t
                copies.append(pltpu.async_copy(
                    pm_hbms[p].at[pl.ds(t * _NPAD + off, _TN)],
                    bm.at[pl.ds(r * _TN, _TN)], sem))
                copies.append(pltpu.async_copy(
                    ps_hbms[p].at[pl.ds(t * _NPAD + off, _TN)],
                    bs.at[pl.ds(r * _TN, _TN)], sem))
        for c in copies:
            c.wait()
        for j in range(_TN // _L):
            om[pl.ds(j * _L, _L)] = bm[pl.ds(j * _L, _L)]
            og[pl.ds(j * _L, _L)] = jnp.zeros((_L,), jnp.float32)

        def mx(t, c):
            for j in range(_TN // _L):
                jo = j * _L
                om[pl.ds(jo, _L)] = jnp.maximum(
                    om[pl.ds(jo, _L)], bm[pl.ds(t * _TN + jo, _L)])
            return c
        lax.fori_loop(1, R, mx, 0)

        def sm(t, c):
            for j in range(_TN // _L):
                jo = j * _L
                m = om[pl.ds(jo, _L)]
                mt = bm[pl.ds(t * _TN + jo, _L)]
                st = bs[pl.ds(t * _TN + jo, _L)]
                og[pl.ds(jo, _L)] = og[pl.ds(jo, _L)] + st * jnp.exp(
                    jnp.maximum(mt - m, -100.0))
            return c
        lax.fori_loop(0, R, sm, 0)

        for j in range(_TN // _L):
            jo = j * _L
            s = og[pl.ds(jo, _L)]
            og[pl.ds(jo, _L)] = jnp.where(s > 0, 1.0 / s, 0.0)
        pltpu.sync_copy(om, gm_hbm.at[pl.ds(off, _TN)])
        pltpu.sync_copy(og, gi_hbm.at[pl.ds(off, _TN)])

    return k


def _make_norm(E):
    ch = E // _NW
    nv = ch // _L
    hw = _NW // 2

    @functools.partial(
        pl.kernel,
        mesh=_mesh(),
        compiler_params=_SC_PARAMS,
        out_type=jax.ShapeDtypeStruct((E,), jnp.float32),
        scratch_types=[
            pltpu.VMEM((ch,), jnp.float32),
            pltpu.VMEM((ch,), jnp.int32),
            pltpu.VMEM((_NPAD,), jnp.float32),
            pltpu.VMEM((_NPAD,), jnp.float32),
            pltpu.VMEM((ch,), jnp.float32),
        ],
    )
    def k(x0_hbm, x1_hbm, idx_hbm, gm_hbm, gi_hbm, out_hbm, xv, iv, gm, gi, ov):
        wid = _wid()
        base = wid * ch

        @pl.when(wid < hw)
        def _():
            pltpu.sync_copy(x0_hbm.at[pl.ds(wid * ch, ch)], xv)

        @pl.when(wid >= hw)
        def _():
            pltpu.sync_copy(x1_hbm.at[pl.ds((wid - hw) * ch, ch)], xv)

        pltpu.sync_copy(idx_hbm.at[pl.ds(base, ch)], iv)
        pltpu.sync_copy(gm_hbm, gm)
        pltpu.sync_copy(gi_hbm, gi)

        @plsc.parallel_loop(0, nv, 1, unroll=4)
        def body(i):
            s = i * _L
            xb = xv[pl.ds(s, _L)]
            ib = iv[pl.ds(s, _L)]
            m = plsc.load_gather(gm, [ib])
            r = plsc.load_gather(gi, [ib])
            ov[pl.ds(s, _L)] = jnp.exp(xb - m) * r
        pltpu.sync_copy(ov, out_hbm.at[pl.ds(base, ch)])

    return k


@jax.jit
def _impl(input, idx, W, b):
    E = input.shape[1]
    H = E // 2
    w2 = W.reshape(1, _D)
    b2 = b.reshape(1, 1)
    i0 = input[0, :H].reshape(H // _EB, _EB, _D)
    i1 = input[0, H:].reshape(H // _EB, _EB, _D)
    x0 = _matvec(i0, w2, b2).reshape(H)
    x1 = _matvec(i1, w2, b2).reshape(H)
    ss = _make_seg_stats(H)
    pm0, ps0 = ss(x0, idx[:H])
    pm1, ps1 = ss(x1, idx[H:])
    gmax, ginv = _make_combine(2)(pm0, ps0, pm1, ps1)
    out = _make_norm(E)(x0, x1, idx, gmax, ginv)
    return out.reshape(1, E, 1)


def kernel(input, idx, W, b):
    return _impl(input, idx, W, b)


# speedup vs baseline: 1.5722x; 1.5722x over previous
import functools

import jax
import jax.numpy as jnp
from jax import lax
from jax.experimental import pallas as pl
from jax.experimental.pallas import tpu as pltpu
from jax.experimental.pallas import tpu_sc as plsc

_D = 128
_N = 10000
_L = 16
_NW = 32
_NPAD = 10240
_TN = _NPAD // _NW
_EB = 4000
_NEG = -3.4028235e38

_GDN = lax.GatherDimensionNumbers(
    offset_dims=(), collapsed_slice_dims=(0,), start_index_map=(0,))


def _take(v, j):
    return lax.gather(v, j[:, None], _GDN, (1,),
                      mode=lax.GatherScatterMode.PROMISE_IN_BOUNDS)


def _mv_body(in_ref, w_ref, b_ref, o_ref):
    blk = in_ref[0]
    y = lax.dot_general(w_ref[...], blk, (((1,), (1,)), ((), ())),
                        preferred_element_type=jnp.float32)
    y = y + b_ref[0, 0]
    o_ref[0] = jnp.where(y >= 0, y, 0.2 * y)


def _matvec(x3, W, b2, nblk, off):
    return pl.pallas_call(
        _mv_body,
        grid=(nblk,),
        in_specs=[
            pl.BlockSpec((1, _EB, _D), lambda i: (i + off, 0, 0)),
            pl.BlockSpec((1, _D), lambda i: (0, 0)),
            pl.BlockSpec((1, 1), lambda i: (0, 0)),
        ],
        out_specs=pl.BlockSpec((1, 1, _EB), lambda i: (i, 0, 0)),
        out_shape=jax.ShapeDtypeStruct((nblk, 1, _EB), jnp.float32),
    )(x3, W, b2)


def _wid():
    return lax.axis_index("s") * 2 + lax.axis_index("c")


def _mesh():
    return plsc.VectorSubcoreMesh(core_axis_name="c", subcore_axis_name="s")


_SC_PARAMS = pltpu.CompilerParams(needs_layout_passes=False)


def _make_seg_stats(E):
    ch = E // _NW
    nv = ch // _L
    assert ch * _NW == E and nv * _L == ch

    @functools.partial(
        pl.kernel,
        mesh=_mesh(),
        compiler_params=_SC_PARAMS,
        out_type=(
            jax.ShapeDtypeStruct((_NW * _NPAD,), jnp.float32),
            jax.ShapeDtypeStruct((_NW * _NPAD,), jnp.float32),
        ),
        scratch_types=[
            pltpu.VMEM((ch,), jnp.float32),
            pltpu.VMEM((ch,), jnp.int32),
            pltpu.VMEM((_NPAD,), jnp.float32),
            pltpu.VMEM((_NPAD,), jnp.float32),
        ],
    )
    def k(x_hbm, idx_hbm, pmax_hbm, psum_hbm, xv, iv, lm, ls):
        wid = _wid()
        base = wid * ch
        pltpu.sync_copy(x_hbm.at[pl.ds(base, ch)], xv)
        pltpu.sync_copy(idx_hbm.at[pl.ds(base, ch)], iv)
        neg = jnp.full((_L,), _NEG, jnp.float32)
        zero = jnp.zeros((_L,), jnp.float32)

        @plsc.parallel_loop(0, _NPAD // _L, 1, unroll=4)
        def init(i):
            lm[pl.ds(i * _L, _L)] = neg
            ls[pl.ds(i * _L, _L)] = zero

        iota = lax.iota(jnp.int32, _L)

        def body(i, c):
            s = i * _L
            xb = xv[pl.ds(s, _L)]
            ib = iv[pl.ds(s, _L)]
            v = xb
            for k_ in (1, 2, 4, 8):
                j = jnp.maximum(iota - k_, 0)
                sh_i = _take(ib, j)
                sh_v = _take(v, j)
                v = jnp.where(sh_i == ib, jnp.maximum(v, sh_v), v)
            nxt = _take(ib, jnp.minimum(iota + 1, _L - 1))
            is_last = (iota == _L - 1) | (ib != nxt)
            z = jnp.where(is_last, (_L - 1) - iota, 0)
            rl = (_L - 1) - lax.rev(plsc.cummax(lax.rev(z, (0,))), (0,))
            m_run = _take(v, rl)
            e = jnp.exp(xb - m_run)
            cs = plsc.cumsum(e)
            cx = cs - e
            prv = _take(ib, jnp.maximum(iota - 1, 0))
            is_start = (iota == 0) | (ib != prv)
            rs = plsc.cummax(jnp.where(is_start, iota, 0))
            run = cs - _take(cx, rs)
            msk = rl == iota
            cur_m = plsc.load_gather(lm, [ib])
            cur_s = plsc.load_gather(ls, [ib])
            nm = jnp.maximum(cur_m, m_run)
            ns = (cur_s * jnp.exp(jnp.maximum(cur_m - nm, -100.0))
                  + run * jnp.exp(m_run - nm))
            plsc.store_scatter(lm, [ib], nm, mask=msk)
            plsc.store_scatter(ls, [ib], ns, mask=msk)
            return c
        lax.fori_loop(0, nv, body, 0, unroll=2)
        pltpu.sync_copy(lm, pmax_hbm.at[pl.ds(wid * _NPAD, _NPAD)])
        pltpu.sync_copy(ls, psum_hbm.at[pl.ds(wid * _NPAD, _NPAD)])

    return k


def _make_combine(nparts):
    R = nparts * _NW

    @functools.partial(
        pl.kernel,
        mesh=_mesh(),
        compiler_params=_SC_PARAMS,
        out_type=(
            jax.ShapeDtypeStruct((_NPAD,), jnp.float32),
            jax.ShapeDtypeStruct((_NPAD,), jnp.float32),
        ),
        scratch_types=[
            pltpu.VMEM((R * _TN,), jnp.float32),
            pltpu.VMEM((R * _TN,), jnp.float32),
            pltpu.VMEM((_TN,), jnp.float32),
            pltpu.VMEM((_TN,), jnp.float32),
            pltpu.SemaphoreType.DMA,
        ],
    )
    def k(*args):
        (pm_hbms, ps_hbms) = (args[0:2 * nparts:2], args[1:2 * nparts:2])
        gm_hbm, gi_hbm, bm, bs, om, og, sem = args[2 * nparts:]
        wid = _wid()
        off = wid * _TN
        for p in range(nparts):
            copies = []
            for t in range(_NW):
                r = p * _NW + t
                copies.append(pltpu.async_copy(
                    pm_hbms[p].at[pl.ds(t * _NPAD + off, _TN)],
                    bm.at[pl.ds(r * _TN, _TN)], sem))
                copies.append(pltpu.async_copy(
                    ps_hbms[p].at[pl.ds(t * _NPAD + off, _TN)],
                    bs.at[pl.ds(r * _TN, _TN)], sem))
            for c in copies:
                c.wait()
        for j in range(_TN // _L):
            om[pl.ds(j * _L, _L)] = bm[pl.ds(j * _L, _L)]
            og[pl.ds(j * _L, _L)] = jnp.zeros((_L,), jnp.float32)

        def mx(t, c):
            for j in range(_TN // _L):
                jo = j * _L
                om[pl.ds(jo, _L)] = jnp.maximum(
                    om[pl.ds(jo, _L)], bm[pl.ds(t * _TN + jo, _L)])
            return c
        lax.fori_loop(1, R, mx, 0)

        def sm(t, c):
            for j in range(_TN // _L):
                jo = j * _L
                m = om[pl.ds(jo, _L)]
                mt = bm[pl.ds(t * _TN + jo, _L)]
                st = bs[pl.ds(t * _TN + jo, _L)]
                og[pl.ds(jo, _L)] = og[pl.ds(jo, _L)] + st * jnp.exp(
                    jnp.maximum(mt - m, -100.0))
            return c
        lax.fori_loop(0, R, sm, 0)

        for j in range(_TN // _L):
            jo = j * _L
            s = og[pl.ds(jo, _L)]
            og[pl.ds(jo, _L)] = jnp.where(s > 0, 1.0 / s, 0.0)
        pltpu.sync_copy(om, gm_hbm.at[pl.ds(off, _TN)])
        pltpu.sync_copy(og, gi_hbm.at[pl.ds(off, _TN)])

    return k


def _make_norm(E):
    ch = E // _NW
    nv = ch // _L

    @functools.partial(
        pl.kernel,
        mesh=_mesh(),
        compiler_params=_SC_PARAMS,
        out_type=jax.ShapeDtypeStruct((E,), jnp.float32),
        scratch_types=[
            pltpu.VMEM((ch,), jnp.float32),
            pltpu.VMEM((ch,), jnp.int32),
            pltpu.VMEM((_NPAD,), jnp.float32),
            pltpu.VMEM((_NPAD,), jnp.float32),
            pltpu.VMEM((ch,), jnp.float32),
        ],
    )
    def k(x_hbm, idx_hbm, gm_hbm, gi_hbm, out_hbm, xv, iv, gm, gi, ov):
        wid = _wid()
        base = wid * ch
        pltpu.sync_copy(x_hbm.at[pl.ds(base, ch)], xv)
        pltpu.sync_copy(idx_hbm.at[pl.ds(base, ch)], iv)
        pltpu.sync_copy(gm_hbm, gm)
        pltpu.sync_copy(gi_hbm, gi)

        @plsc.parallel_loop(0, nv, 1, unroll=4)
        def body(i):
            s = i * _L
            xb = xv[pl.ds(s, _L)]
            ib = iv[pl.ds(s, _L)]
            m = plsc.load_gather(gm, [ib])
            r = plsc.load_gather(gi, [ib])
            ov[pl.ds(s, _L)] = jnp.exp(xb - m) * r
        pltpu.sync_copy(ov, out_hbm.at[pl.ds(base, ch)])

    return k


@jax.jit
def _impl(input, idx, W, b):
    E = input.shape[1]
    H0 = 192000
    H1 = E - H0
    nb0 = H0 // _EB
    nb1 = H1 // _EB
    w2 = W.reshape(1, _D)
    b2 = b.reshape(1, 1)
    x3 = input.reshape(E // _EB, _EB, _D)
    x0 = _matvec(x3, w2, b2, nb0, 0).reshape(H0)
    x1 = _matvec(x3, w2, b2, nb1, nb0).reshape(H1)
    pm0, ps0 = _make_seg_stats(H0)(x0, idx[:H0])
    pm1, ps1 = _make_seg_stats(H1)(x1, idx[H0:])
    gmax, ginv = _make_combine(2)(pm0, ps0, pm1, ps1)
    x = jnp.concatenate([x0, x1])
    out = _make_norm(E)(x, idx, gmax, ginv)
    return out.reshape(1, E, 1)


def kernel(input, idx, W, b):
    return _impl(input, idx, W, b)


# speedup vs baseline: 1.6836x; 1.0708x over previous
import functools

import jax
import jax.numpy as jnp
from jax import lax
from jax.experimental import pallas as pl
from jax.experimental.pallas import tpu as pltpu
from jax.experimental.pallas import tpu_sc as plsc

_D = 128
_N = 10000
_L = 16
_NW = 32
_NPAD = 10240
_TN = _NPAD // _NW
_EB = 4000
_NEG = -3.4028235e38

_GDN = lax.GatherDimensionNumbers(
    offset_dims=(), collapsed_slice_dims=(0,), start_index_map=(0,))


def _take(v, j):
    return lax.gather(v, j[:, None], _GDN, (1,),
                      mode=lax.GatherScatterMode.PROMISE_IN_BOUNDS)


def _mv_body(in_ref, w_ref, b_ref, o_ref):
    blk = in_ref[0]
    y = lax.dot_general(w_ref[...], blk, (((1,), (1,)), ((), ())),
                        preferred_element_type=jnp.float32)
    y = y + b_ref[0, 0]
    o_ref[0] = jnp.where(y >= 0, y, 0.2 * y)


def _matvec(x3, W, b2):
    nblk = x3.shape[0]
    return pl.pallas_call(
        _mv_body,
        grid=(nblk,),
        in_specs=[
            pl.BlockSpec((1, _EB, _D), lambda i: (i, 0, 0)),
            pl.BlockSpec((1, _D), lambda i: (0, 0)),
            pl.BlockSpec((1, 1), lambda i: (0, 0)),
        ],
        out_specs=pl.BlockSpec((1, 1, _EB), lambda i: (i, 0, 0)),
        out_shape=jax.ShapeDtypeStruct((nblk, 1, _EB), jnp.float32),
    )(x3, W, b2)


def _wid():
    return lax.axis_index("s") * 2 + lax.axis_index("c")


def _mesh():
    return plsc.VectorSubcoreMesh(core_axis_name="c", subcore_axis_name="s")


_SC_PARAMS = pltpu.CompilerParams(needs_layout_passes=False)


def _make_seg_stats(E):
    ch = E // _NW
    nv = ch // _L

    @functools.partial(
        pl.kernel,
        mesh=_mesh(),
        compiler_params=_SC_PARAMS,
        out_type=(
            jax.ShapeDtypeStruct((_NW * _NPAD,), jnp.float32),
            jax.ShapeDtypeStruct((_NW * _NPAD,), jnp.float32),
        ),
        scratch_types=[
            pltpu.VMEM((ch,), jnp.float32),
            pltpu.VMEM((ch,), jnp.int32),
            pltpu.VMEM((_NPAD,), jnp.float32),
            pltpu.VMEM((_NPAD,), jnp.float32),
        ],
    )
    def k(x_hbm, idx_hbm, pmax_hbm, psum_hbm, xv, iv, lm, ls):
        wid = _wid()
        base = wid * ch
        pltpu.sync_copy(x_hbm.at[pl.ds(base, ch)], xv)
        pltpu.sync_copy(idx_hbm.at[pl.ds(base, ch)], iv)
        neg = jnp.full((_L,), _NEG, jnp.float32)
        zero = jnp.zeros((_L,), jnp.float32)

        @plsc.parallel_loop(0, _NPAD // _L, 1, unroll=4)
        def init(i):
            lm[pl.ds(i * _L, _L)] = neg
            ls[pl.ds(i * _L, _L)] = zero

        iota = lax.iota(jnp.int32, _L)

        def body(i, c):
            s = i * _L
            xb = xv[pl.ds(s, _L)]
            ib = iv[pl.ds(s, _L)]
            v = xb
            for k_ in (1, 2, 4, 8):
                j = jnp.maximum(iota - k_, 0)
                sh_i = _take(ib, j)
                sh_v = _take(v, j)
                v = jnp.where(sh_i == ib, jnp.maximum(v, sh_v), v)
            nxt = _take(ib, jnp.minimum(iota + 1, _L - 1))
            is_last = (iota == _L - 1) | (ib != nxt)
            z = jnp.where(is_last, (_L - 1) - iota, 0)
            rl = (_L - 1) - lax.rev(plsc.cummax(lax.rev(z, (0,))), (0,))
            m_run = _take(v, rl)
            e = jnp.exp(xb - m_run)
            cs = plsc.cumsum(e)
            cx = cs - e
            prv = _take(ib, jnp.maximum(iota - 1, 0))
            is_start = (iota == 0) | (ib != prv)
            rs = plsc.cummax(jnp.where(is_start, iota, 0))
            run = cs - _take(cx, rs)
            msk = rl == iota
            cur_m = plsc.load_gather(lm, [ib])
            cur_s = plsc.load_gather(ls, [ib])
            nm = jnp.maximum(cur_m, m_run)
            ns = (cur_s * jnp.exp(jnp.maximum(cur_m - nm, -100.0))
                  + run * jnp.exp(m_run - nm))
            plsc.store_scatter(lm, [ib], nm, mask=msk)
            plsc.store_scatter(ls, [ib], ns, mask=msk)
            return c
        lax.fori_loop(0, nv, body, 0, unroll=4)
        pltpu.sync_copy(lm, pmax_hbm.at[pl.ds(wid * _NPAD, _NPAD)])
        pltpu.sync_copy(ls, psum_hbm.at[pl.ds(wid * _NPAD, _NPAD)])

    return k


def _make_combine():
    @functools.partial(
        pl.kernel,
        mesh=_mesh(),
        compiler_params=_SC_PARAMS,
        out_type=(
            jax.ShapeDtypeStruct((_NPAD,), jnp.float32),
            jax.ShapeDtypeStruct((_NPAD,), jnp.float32),
        ),
        scratch_types=[
            pltpu.VMEM((_NW * _TN,), jnp.float32),
            pltpu.VMEM((_NW * _TN,), jnp.float32),
            pltpu.VMEM((_TN,), jnp.float32),
            pltpu.VMEM((_TN,), jnp.float32),
            pltpu.SemaphoreType.DMA,
        ],
    )
    def k(pm_hbm, ps_hbm, gm_hbm, gi_hbm, bm, bs, om, og, sem):
        wid = _wid()
        off = wid * _TN
        copies = []
        for t in range(_NW):
            copies.append(pltpu.async_copy(
                pm_hbm.at[pl.ds(t * _NPAD + off, _TN)],
                bm.at[pl.ds(t * _TN, _TN)], sem))
            copies.append(pltpu.async_copy(
                ps_hbm.at[pl.ds(t * _NPAD + off, _TN)],
                bs.at[pl.ds(t * _TN, _TN)], sem))
        for c in copies:
            c.wait()
        for j in range(_TN // _L):
            m = bm[pl.ds(j * _L, _L)]
            for t in range(1, _NW):
                m = jnp.maximum(m, bm[pl.ds(t * _TN + j * _L, _L)])
            s = jnp.zeros((_L,), jnp.float32)
            for t in range(_NW):
                mt = bm[pl.ds(t * _TN + j * _L, _L)]
                st = bs[pl.ds(t * _TN + j * _L, _L)]
                s = s + st * jnp.exp(jnp.maximum(mt - m, -100.0))
            om[pl.ds(j * _L, _L)] = m
            og[pl.ds(j * _L, _L)] = jnp.where(s > 0, 1.0 / s, 0.0)
        pltpu.sync_copy(om, gm_hbm.at[pl.ds(off, _TN)])
        pltpu.sync_copy(og, gi_hbm.at[pl.ds(off, _TN)])

    return k


def _make_norm(E):
    ch = E // _NW
    nv = ch // _L

    @functools.partial(
        pl.kernel,
        mesh=_mesh(),
        compiler_params=_SC_PARAMS,
        out_type=jax.ShapeDtypeStruct((E,), jnp.float32),
        scratch_types=[
            pltpu.VMEM((ch,), jnp.float32),
            pltpu.VMEM((ch,), jnp.int32),
            pltpu.VMEM((_NPAD,), jnp.float32),
            pltpu.VMEM((_NPAD,), jnp.float32),
            pltpu.VMEM((ch,), jnp.float32),
        ],
    )
    def k(x_hbm, idx_hbm, gm_hbm, gi_hbm, out_hbm, xv, iv, gm, gi, ov):
        wid = _wid()
        base = wid * ch
        pltpu.sync_copy(x_hbm.at[pl.ds(base, ch)], xv)
        pltpu.sync_copy(idx_hbm.at[pl.ds(base, ch)], iv)
        pltpu.sync_copy(gm_hbm, gm)
        pltpu.sync_copy(gi_hbm, gi)

        @plsc.parallel_loop(0, nv, 1, unroll=4)
        def body(i):
            s = i * _L
            xb = xv[pl.ds(s, _L)]
            ib = iv[pl.ds(s, _L)]
            m = plsc.load_gather(gm, [ib])
            r = plsc.load_gather(gi, [ib])
            ov[pl.ds(s, _L)] = jnp.exp(xb - m) * r
        pltpu.sync_copy(ov, out_hbm.at[pl.ds(base, ch)])

    return k


@jax.jit
def _impl(input, idx, W, b):
    E = input.shape[1]
    x3 = input.reshape(E // _EB, _EB, _D)
    x = _matvec(x3, W.reshape(1, _D), b.reshape(1, 1)).reshape(E)
    pmax, psum = _make_seg_stats(E)(x, idx)
    gmax, ginv = _make_combine()(pmax, psum)
    out = _make_norm(E)(x, idx, gmax, ginv)
    return out.reshape(1, E, 1)


def kernel(input, idx, W, b):
    return _impl(input, idx, W, b)


# speedup vs baseline: 1.7036x; 1.0119x over previous
import functools

import jax
import jax.numpy as jnp
from jax import lax
from jax.experimental import pallas as pl
from jax.experimental.pallas import tpu as pltpu
from jax.experimental.pallas import tpu_sc as plsc

_D = 128
_N = 10000
_L = 16
_NW = 32
_NPAD = 10240
_TN = _NPAD // _NW
_EB = 4000
_NEG = -3.4028235e38

_GDN = lax.GatherDimensionNumbers(
    offset_dims=(), collapsed_slice_dims=(0,), start_index_map=(0,))


def _take(v, j):
    return lax.gather(v, j[:, None], _GDN, (1,),
                      mode=lax.GatherScatterMode.PROMISE_IN_BOUNDS)


def _mv_body(in_ref, w_ref, b_ref, o_ref):
    blk = in_ref[0]
    y = lax.dot_general(w_ref[...], blk, (((1,), (1,)), ((), ())),
                        preferred_element_type=jnp.float32)
    y = y + b_ref[0, 0]
    o_ref[0] = jnp.where(y >= 0, y, 0.2 * y)


def _matvec(x3, W, b2):
    nblk = x3.shape[0]
    return pl.pallas_call(
        _mv_body,
        grid=(nblk,),
        in_specs=[
            pl.BlockSpec((1, _EB, _D), lambda i: (i, 0, 0)),
            pl.BlockSpec((1, _D), lambda i: (0, 0)),
            pl.BlockSpec((1, 1), lambda i: (0, 0)),
        ],
        out_specs=pl.BlockSpec((1, 1, _EB), lambda i: (i, 0, 0)),
        out_shape=jax.ShapeDtypeStruct((nblk, 1, _EB), jnp.float32),
    )(x3, W, b2)


def _wid():
    return lax.axis_index("s") * 2 + lax.axis_index("c")


def _mesh():
    return plsc.VectorSubcoreMesh(core_axis_name="c", subcore_axis_name="s")


_SC_PARAMS = pltpu.CompilerParams(needs_layout_passes=False)


def _make_seg_stats(E):
    ch = E // _NW
    nv = ch // _L

    @functools.partial(
        pl.kernel,
        mesh=_mesh(),
        compiler_params=_SC_PARAMS,
        out_type=(
            jax.ShapeDtypeStruct((_NW * _NPAD,), jnp.float32),
            jax.ShapeDtypeStruct((_NW * _NPAD,), jnp.float32),
        ),
        scratch_types=[
            pltpu.VMEM((ch,), jnp.float32),
            pltpu.VMEM((ch,), jnp.int32),
            pltpu.VMEM((_NPAD,), jnp.float32),
            pltpu.VMEM((_NPAD,), jnp.float32),
        ],
    )
    def k(x_hbm, idx_hbm, pmax_hbm, psum_hbm, xv, iv, lm, ls):
        wid = _wid()
        base = wid * ch
        pltpu.sync_copy(x_hbm.at[pl.ds(base, ch)], xv)
        pltpu.sync_copy(idx_hbm.at[pl.ds(base, ch)], iv)
        neg = jnp.full((_L,), _NEG, jnp.float32)
        zero = jnp.zeros((_L,), jnp.float32)

        @plsc.parallel_loop(0, _NPAD // _L, 1, unroll=4)
        def init(i):
            lm[pl.ds(i * _L, _L)] = neg
            ls[pl.ds(i * _L, _L)] = zero

        iota = lax.iota(jnp.int32, _L)

        def body(i, c):
            s = i * _L
            xb = xv[pl.ds(s, _L)]
            ib = iv[pl.ds(s, _L)]
            v = xb
            for k_ in (1, 2, 4, 8):
                j = jnp.maximum(iota - k_, 0)
                sh_i = _take(ib, j)
                sh_v = _take(v, j)
                v = jnp.where(sh_i == ib, jnp.maximum(v, sh_v), v)
            nxt = _take(ib, jnp.minimum(iota + 1, _L - 1))
            is_last = (iota == _L - 1) | (ib != nxt)
            z = jnp.where(is_last, (_L - 1) - iota, 0)
            rl = (_L - 1) - lax.rev(plsc.cummax(lax.rev(z, (0,))), (0,))
            m_run = _take(v, rl)
            e = jnp.exp(xb - m_run)
            cs = plsc.cumsum(e)
            cx = cs - e
            prv = _take(ib, jnp.maximum(iota - 1, 0))
            is_start = (iota == 0) | (ib != prv)
            rs = plsc.cummax(jnp.where(is_start, iota, 0))
            run = cs - _take(cx, rs)
            msk = rl == iota
            cur_m = plsc.load_gather(lm, [ib])
            cur_s = plsc.load_gather(ls, [ib])
            nm = jnp.maximum(cur_m, m_run)
            ns = (cur_s * jnp.exp(jnp.maximum(cur_m - nm, -100.0))
                  + run * jnp.exp(m_run - nm))
            plsc.store_scatter(lm, [ib], nm, mask=msk)
            plsc.store_scatter(ls, [ib], ns, mask=msk)
            return c
        lax.fori_loop(0, nv, body, 0, unroll=2)
        pltpu.sync_copy(lm, pmax_hbm.at[pl.ds(wid * _NPAD, _NPAD)])
        pltpu.sync_copy(ls, psum_hbm.at[pl.ds(wid * _NPAD, _NPAD)])

    return k


def _make_combine():
    @functools.partial(
        pl.kernel,
        mesh=_mesh(),
        compiler_params=_SC_PARAMS,
        out_type=(
            jax.ShapeDtypeStruct((_NPAD,), jnp.float32),
            jax.ShapeDtypeStruct((_NPAD,), jnp.float32),
        ),
        scratch_types=[
            pltpu.VMEM((_NW * _TN,), jnp.float32),
            pltpu.VMEM((_NW * _TN,), jnp.float32),
            pltpu.VMEM((_TN,), jnp.float32),
            pltpu.VMEM((_TN,), jnp.float32),
            pltpu.SemaphoreType.DMA,
        ],
    )
    def k(pm_hbm, ps_hbm, gm_hbm, gi_hbm, bm, bs, om, og, sem):
        wid = _wid()
        off = wid * _TN
        copies = []
        for t in range(_NW):
            copies.append(pltpu.async_copy(
                pm_hbm.at[pl.ds(t * _NPAD + off, _TN)],
                bm.at[pl.ds(t * _TN, _TN)], sem))
            copies.append(pltpu.async_copy(
                ps_hbm.at[pl.ds(t * _NPAD + off, _TN)],
                bs.at[pl.ds(t * _TN, _TN)], sem))
        for c in copies:
            c.wait()
        for j in range(_TN // _L):
            m = bm[pl.ds(j * _L, _L)]
            for t in range(1, _NW):
                m = jnp.maximum(m, bm[pl.ds(t * _TN + j * _L, _L)])
            s = jnp.zeros((_L,), jnp.float32)
            for t in range(_NW):
                mt = bm[pl.ds(t * _TN + j * _L, _L)]
                st = bs[pl.ds(t * _TN + j * _L, _L)]
                s = s + st * jnp.exp(jnp.maximum(mt - m, -100.0))
            om[pl.ds(j * _L, _L)] = m
            og[pl.ds(j * _L, _L)] = jnp.where(s > 0, 1.0 / s, 0.0)
        pltpu.sync_copy(om, gm_hbm.at[pl.ds(off, _TN)])
        pltpu.sync_copy(og, gi_hbm.at[pl.ds(off, _TN)])

    return k


def _make_norm(E):
    ch = E // _NW
    nv = ch // _L

    @functools.partial(
        pl.kernel,
        mesh=_mesh(),
        compiler_params=_SC_PARAMS,
        out_type=jax.ShapeDtypeStruct((E,), jnp.float32),
        scratch_types=[
            pltpu.VMEM((ch,), jnp.float32),
            pltpu.VMEM((ch,), jnp.int32),
            pltpu.VMEM((_NPAD,), jnp.float32),
            pltpu.VMEM((_NPAD,), jnp.float32),
            pltpu.VMEM((ch,), jnp.float32),
        ],
    )
    def k(x_hbm, idx_hbm, gm_hbm, gi_hbm, out_hbm, xv, iv, gm, gi, ov):
        wid = _wid()
        base = wid * ch
        pltpu.sync_copy(x_hbm.at[pl.ds(base, ch)], xv)
        pltpu.sync_copy(idx_hbm.at[pl.ds(base, ch)], iv)
        pltpu.sync_copy(gm_hbm, gm)
        pltpu.sync_copy(gi_hbm, gi)

        @plsc.parallel_loop(0, nv, 1, unroll=4)
        def body(i):
            s = i * _L
            xb = xv[pl.ds(s, _L)]
            ib = iv[pl.ds(s, _L)]
            m = plsc.load_gather(gm, [ib])
            r = plsc.load_gather(gi, [ib])
            ov[pl.ds(s, _L)] = jnp.exp(xb - m) * r
        pltpu.sync_copy(ov, out_hbm.at[pl.ds(base, ch)])

    return k


@jax.jit
def _impl(input, idx, W, b):
    E = input.shape[1]
    x3 = input.reshape(E // _EB, _EB, _D)
    x = _matvec(x3, W.reshape(1, _D), b.reshape(1, 1)).reshape(E)
    pmax, psum = _make_seg_stats(E)(x, idx)
    gmax, ginv = _make_combine()(pmax, psum)
    out = _make_norm(E)(x, idx, gmax, ginv)
    return out.reshape(1, E, 1)


def kernel(input, idx, W, b):
    return _impl(input, idx, W, b)
